# baseline (speedup 1.0000x reference)
import jax
import jax.numpy as jnp
from jax import lax
from jax.experimental import pallas as pl
from jax.experimental.pallas import tpu as pltpu
from jax.experimental.pallas import tpu_sc as plsc

BATCH = 1024
INPUT_DIM = 100000
NUM_RULES = 32768
LANES = 16
NUM_CORES = 2
NUM_SUBCORES = 16
NUM_WORKERS = NUM_CORES * NUM_SUBCORES
NUM_HALVES = 2
HALF_RULES = NUM_RULES // NUM_HALVES
RULES_PER_W = HALF_RULES // NUM_WORKERS
KR = 16
NCHUNK = RULES_PER_W // KR
GROUPS = BATCH // LANES


def _sc_body(ft_hbm, idx1_hbm, idx2_hbm, w_hbm, out_hbm,
             i1_v, i2_v, ws_v, scale_v, f1_v, f2_v, o_v, sem_g, sem_o):
    wid = lax.axis_index("s") * NUM_CORES + lax.axis_index("c")
    base = wid * RULES_PER_W

    pltpu.sync_copy(idx1_hbm.at[pl.ds(base, RULES_PER_W)], i1_v)
    pltpu.sync_copy(idx2_hbm.at[pl.ds(base, RULES_PER_W)], i2_v)
    pltpu.sync_copy(w_hbm.at[pl.ds(base, RULES_PER_W)], ws_v)

    @plsc.parallel_loop(0, RULES_PER_W // LANES, unroll=4)
    def s_body(k):
        b = k * LANES
        w = ws_v[pl.ds(b, LANES)]
        scale_v[pl.ds(b, LANES)] = 1.0 / (1.0 + jnp.exp(-w))

    def start_gathers(c, slot):
        i1vec = i1_v[pl.ds(c * KR, KR)]
        i2vec = i2_v[pl.ds(c * KR, KR)]
        pltpu.make_async_copy(ft_hbm.at[i1vec], f1_v[slot], sem_g[slot]).start()
        pltpu.make_async_copy(ft_hbm.at[i2vec], f2_v[slot], sem_g[slot]).start()

    def wait_gathers(slot):
        zeros = i1_v[pl.ds(0, KR)]
        pltpu.make_async_copy(ft_hbm.at[zeros], f1_v[slot], sem_g[slot]).wait()
        pltpu.make_async_copy(ft_hbm.at[zeros], f2_v[slot], sem_g[slot]).wait()

    def wait_out(slot):
        pltpu.make_async_copy(
            o_v[slot], out_hbm.at[pl.ds(0, KR), :], sem_o[slot]).wait()

    start_gathers(0, 0)

    def pair_body(t, carry):
        for sub in range(2):
            c = t * 2 + sub
            slot = sub
            wait_gathers(slot)
            nxt = lax.rem(c + 1, NCHUNK)
            start_gathers(nxt, 1 - slot)

            @pl.when(c >= 2)
            def _():
                wait_out(slot)

            sv = scale_v[pl.ds(c * KR, KR)]
            for rl in range(KR):
                sbc = jax.lax.broadcast(sv[rl], (LANES,))

                @plsc.parallel_loop(0, GROUPS, unroll=4)
                def g_body(g, rl=rl, slot=slot, sbc=sbc):
                    b = g * LANES
                    v1 = f1_v[slot][rl, pl.ds(b, LANES)]
                    v2 = f2_v[slot][rl, pl.ds(b, LANES)]
                    o_v[slot][rl, pl.ds(b, LANES)] = sbc * (v1 * v2)
            pltpu.make_async_copy(
                o_v[slot], out_hbm.at[pl.ds(base + c * KR, KR), :],
                sem_o[slot]).start()
        return carry

    lax.fori_loop(0, NCHUNK // 2, pair_body, 0)

    wait_gathers(0)
    for slot in range(2):
        wait_out(slot)


def kernel(facts, idx, weight):
    idx32 = idx.astype(jnp.int32)
    idx1 = idx32[:, 0]
    idx2 = idx32[:, 1]
    ft = facts.T
    mesh = plsc.VectorSubcoreMesh(core_axis_name="c", subcore_axis_name="s")
    f = pl.kernel(
        _sc_body,
        out_type=jax.ShapeDtypeStruct((HALF_RULES, BATCH), jnp.float32),
        mesh=mesh,
        compiler_params=pltpu.CompilerParams(needs_layout_passes=False),
        scratch_types=[
            pltpu.VMEM((RULES_PER_W,), jnp.int32),
            pltpu.VMEM((RULES_PER_W,), jnp.int32),
            pltpu.VMEM((RULES_PER_W,), jnp.float32),
            pltpu.VMEM((RULES_PER_W,), jnp.float32),
            [pltpu.VMEM((KR, BATCH), jnp.float32)] * 2,
            [pltpu.VMEM((KR, BATCH), jnp.float32)] * 2,
            [pltpu.VMEM((KR, BATCH), jnp.float32)] * 2,
            [pltpu.SemaphoreType.DMA] * 2,
            [pltpu.SemaphoreType.DMA] * 2,
        ],
    )
    halves = []
    for h in range(NUM_HALVES):
        sl = slice(h * HALF_RULES, (h + 1) * HALF_RULES)
        out_t = f(ft, idx1[sl], idx2[sl], weight[sl])
        halves.append(out_t.T)
    return jnp.concatenate(halves, axis=1)


# speedup vs baseline: 1.3442x; 1.3442x over previous
import jax
import jax.numpy as jnp
from jax import lax
from jax.experimental import pallas as pl
from jax.experimental.pallas import tpu as pltpu
from jax.experimental.pallas import tpu_sc as plsc

BATCH = 1024
INPUT_DIM = 100000
NUM_RULES = 32768
LANES = 16
NUM_CORES = 2
NUM_SUBCORES = 16
NUM_WORKERS = NUM_CORES * NUM_SUBCORES
RULES_PER_W = NUM_RULES // NUM_WORKERS
KR = 16
NCHUNK = RULES_PER_W // KR
GROUPS = BATCH // LANES


def _sc_body(ft_hbm, idx1_hbm, idx2_hbm, w_hbm, out_hbm,
             i1_v, i2_v, ws_v, scale_v, f1_v, f2_v, o_v, sem_g, sem_o):
    wid = lax.axis_index("s") * NUM_CORES + lax.axis_index("c")
    base = wid * RULES_PER_W

    pltpu.sync_copy(idx1_hbm.at[pl.ds(base, RULES_PER_W)], i1_v)
    pltpu.sync_copy(idx2_hbm.at[pl.ds(base, RULES_PER_W)], i2_v)
    pltpu.sync_copy(w_hbm.at[pl.ds(base, RULES_PER_W)], ws_v)

    @plsc.parallel_loop(0, RULES_PER_W // LANES, unroll=4)
    def s_body(k):
        b = k * LANES
        w = ws_v[pl.ds(b, LANES)]
        scale_v[pl.ds(b, LANES)] = 1.0 / (1.0 + jnp.exp(-w))

    def start_gathers(c, slot):
        i1vec = i1_v[pl.ds(c * KR, KR)]
        i2vec = i2_v[pl.ds(c * KR, KR)]
        pltpu.make_async_copy(ft_hbm.at[i1vec], f1_v[slot], sem_g[slot]).start()
        pltpu.make_async_copy(ft_hbm.at[i2vec], f2_v[slot], sem_g[slot]).start()

    def wait_gathers(slot):
        zeros = i1_v[pl.ds(0, KR)]
        pltpu.make_async_copy(ft_hbm.at[zeros], f1_v[slot], sem_g[slot]).wait()
        pltpu.make_async_copy(ft_hbm.at[zeros], f2_v[slot], sem_g[slot]).wait()

    def wait_out(slot):
        pltpu.make_async_copy(
            o_v[slot], out_hbm.at[pl.ds(0, KR), :], sem_o[slot]).wait()

    start_gathers(0, 0)

    def pair_body(t, carry):
        for sub in range(2):
            c = t * 2 + sub
            slot = sub
            wait_gathers(slot)
            nxt = lax.rem(c + 1, NCHUNK)
            start_gathers(nxt, 1 - slot)

            @pl.when(c >= 2)
            def _():
                wait_out(slot)

            sv = scale_v[pl.ds(c * KR, KR)]
            for rl in range(KR):
                sbc = jax.lax.broadcast(sv[rl], (LANES,))

                @plsc.parallel_loop(0, GROUPS, unroll=4)
                def g_body(g, rl=rl, slot=slot, sbc=sbc):
                    b = g * LANES
                    v1 = f1_v[slot][rl, pl.ds(b, LANES)]
                    v2 = f2_v[slot][rl, pl.ds(b, LANES)]
                    o_v[slot][rl, pl.ds(b, LANES)] = sbc * (v1 * v2)
            pltpu.make_async_copy(
                o_v[slot], out_hbm.at[pl.ds(base + c * KR, KR), :],
                sem_o[slot]).start()
        return carry

    lax.fori_loop(0, NCHUNK // 2, pair_body, 0)

    wait_gathers(0)
    for slot in range(2):
        wait_out(slot)


def kernel(facts, idx, weight):
    idx32 = idx.astype(jnp.int32)
    idx1 = idx32[:, 0]
    idx2 = idx32[:, 1]
    ft = facts.T
    mesh = plsc.VectorSubcoreMesh(core_axis_name="c", subcore_axis_name="s")
    f = pl.kernel(
        _sc_body,
        out_type=jax.ShapeDtypeStruct((NUM_RULES, BATCH), jnp.float32),
        mesh=mesh,
        compiler_params=pltpu.CompilerParams(needs_layout_passes=False),
        scratch_types=[
            pltpu.VMEM((RULES_PER_W,), jnp.int32),
            pltpu.VMEM((RULES_PER_W,), jnp.int32),
            pltpu.VMEM((RULES_PER_W,), jnp.float32),
            pltpu.VMEM((RULES_PER_W,), jnp.float32),
            [pltpu.VMEM((KR, BATCH), jnp.float32)] * 2,
            [pltpu.VMEM((KR, BATCH), jnp.float32)] * 2,
            [pltpu.VMEM((KR, BATCH), jnp.float32)] * 2,
            [pltpu.SemaphoreType.DMA] * 2,
            [pltpu.SemaphoreType.DMA] * 2,
        ],
    )
    out_t = f(ft, idx1, idx2, weight)
    return out_t.T
